# baseline (speedup 1.0000x reference)
import numpy as np

import jax
import jax.numpy as jnp
from jax.experimental import pallas as pl
from jax.experimental.pallas import tpu as pltpu

_B = 128
_V = 100000
_ROWS_PER_BLOCK = 8


def _gumbel_weights_np(n):
    i = np.arange(n, dtype=np.uint32)
    k1 = np.uint32(0)
    k2 = np.uint32(42)
    k3 = k1 ^ k2 ^ np.uint32(0x1BD11BDA)
    ks = (k1, k2, k3)
    rot_a = (13, 15, 26, 6)
    rot_b = (17, 29, 16, 24)

    def rotl(x, d):
        return (x << np.uint32(d)) | (x >> np.uint32(32 - d))

    def four_rounds(x0, x1, rots):
        for r in rots:
            x0 = x0 + x1
            x1 = x0 ^ rotl(x1, r)
        return x0, x1

    with np.errstate(over="ignore"):
        x0 = np.zeros(n, np.uint32) + ks[0]
        x1 = i + ks[1]
        x0, x1 = four_rounds(x0, x1, rot_a)
        x0 = x0 + ks[1]
        x1 = x1 + ks[2] + np.uint32(1)
        x0, x1 = four_rounds(x0, x1, rot_b)
        x0 = x0 + ks[2]
        x1 = x1 + ks[0] + np.uint32(2)
        x0, x1 = four_rounds(x0, x1, rot_a)
        x0 = x0 + ks[0]
        x1 = x1 + ks[1] + np.uint32(3)
        x0, x1 = four_rounds(x0, x1, rot_b)
        x0 = x0 + ks[1]
        x1 = x1 + ks[2] + np.uint32(4)
        x0, x1 = four_rounds(x0, x1, rot_a)
        x0 = x0 + ks[2]
        x1 = x1 + ks[0] + np.uint32(5)
    bits = x0 ^ x1

    tiny = np.float32(np.finfo(np.float32).tiny)
    fb = (bits >> np.uint32(9)) | np.uint32(0x3F800000)
    f = fb.view(np.float32) - np.float32(1.0)
    u = np.maximum(tiny, f * (np.float32(1.0) - tiny) + tiny)
    w = -1.0 / np.log(u.astype(np.float64))
    return w.astype(np.float32)


_W = _gumbel_weights_np(_B * _V).reshape(_B, _V)


def _sample_kernel(probs_ref, w_ref, p_ref, act_ref):
    x = probs_ref[...]

    s = jnp.sum(x, axis=1, keepdims=True)
    s = jnp.where(s == 0.0, 1.0, s)
    p_ref[...] = x * (1.0 / s)

    t = x * w_ref[...]
    tmax = jnp.max(t, axis=1, keepdims=True)
    ci = jax.lax.broadcasted_iota(jnp.int32, x.shape, 1)
    cand = jnp.where(t == tmax, ci, jnp.int32(_V))
    act_ref[...] = jnp.min(cand, axis=1, keepdims=True)


@jax.jit
def kernel(probs):
    grid = (_B // _ROWS_PER_BLOCK,)
    p3d, act = pl.pallas_call(
        _sample_kernel,
        grid=grid,
        in_specs=[
            pl.BlockSpec((_ROWS_PER_BLOCK, None, _V), lambda i: (i, 0, 0)),
            pl.BlockSpec((_ROWS_PER_BLOCK, _V), lambda i: (i, 0)),
        ],
        out_specs=[
            pl.BlockSpec((_ROWS_PER_BLOCK, None, _V), lambda i: (i, 0, 0)),
            pl.BlockSpec((_ROWS_PER_BLOCK, 1), lambda i: (i, 0)),
        ],
        out_shape=[
            jax.ShapeDtypeStruct((_B, 1, _V), jnp.float32),
            jax.ShapeDtypeStruct((_B, 1), jnp.int32),
        ],
        compiler_params=pltpu.CompilerParams(
            dimension_semantics=("parallel",),
        ),
    )(probs, jnp.asarray(_W))
    return p3d, act


# speedup vs baseline: 1.9788x; 1.9788x over previous
import numpy as np

import jax
import jax.numpy as jnp
from jax.experimental import pallas as pl
from jax.experimental.pallas import tpu as pltpu

_B = 128
_V = 100000
_ROWS_PER_BLOCK = 8


def _gumbel_weights_np(n):
    i = np.arange(n, dtype=np.uint32)
    k1 = np.uint32(0)
    k2 = np.uint32(42)
    k3 = k1 ^ k2 ^ np.uint32(0x1BD11BDA)
    ks = (k1, k2, k3)
    rot_a = (13, 15, 26, 6)
    rot_b = (17, 29, 16, 24)

    def rotl(x, d):
        return (x << np.uint32(d)) | (x >> np.uint32(32 - d))

    def four_rounds(x0, x1, rots):
        for r in rots:
            x0 = x0 + x1
            x1 = x0 ^ rotl(x1, r)
        return x0, x1

    with np.errstate(over="ignore"):
        x0 = np.zeros(n, np.uint32) + ks[0]
        x1 = i + ks[1]
        x0, x1 = four_rounds(x0, x1, rot_a)
        x0 = x0 + ks[1]
        x1 = x1 + ks[2] + np.uint32(1)
        x0, x1 = four_rounds(x0, x1, rot_b)
        x0 = x0 + ks[2]
        x1 = x1 + ks[0] + np.uint32(2)
        x0, x1 = four_rounds(x0, x1, rot_a)
        x0 = x0 + ks[0]
        x1 = x1 + ks[1] + np.uint32(3)
        x0, x1 = four_rounds(x0, x1, rot_b)
        x0 = x0 + ks[1]
        x1 = x1 + ks[2] + np.uint32(4)
        x0, x1 = four_rounds(x0, x1, rot_a)
        x0 = x0 + ks[2]
        x1 = x1 + ks[0] + np.uint32(5)
    bits = x0 ^ x1

    tiny = np.float32(np.finfo(np.float32).tiny)
    fb = (bits >> np.uint32(9)) | np.uint32(0x3F800000)
    f = fb.view(np.float32) - np.float32(1.0)
    u = np.maximum(tiny, f * (np.float32(1.0) - tiny) + tiny)
    w = -1.0 / np.log(u.astype(np.float64))
    return w.astype(np.float32)


_W = _gumbel_weights_np(_B * _V).reshape(_B, _V)


def _sample_kernel(probs_ref, w_ref, p_ref, act_ref):
    x = probs_ref[...]

    s = jnp.sum(x, axis=1, keepdims=True)
    s = jnp.where(s == 0.0, 1.0, s)
    p_ref[...] = x * (1.0 / s)

    t = x * w_ref[...]
    tmax = jnp.max(t, axis=1, keepdims=True)
    ci = jax.lax.broadcasted_iota(jnp.int32, x.shape, 1)
    cand = jnp.where(t == tmax, ci, jnp.int32(_V))
    act_ref[...] = jnp.min(cand, axis=1, keepdims=True)


@jax.jit
def _run(probs, w):
    x2d = probs.reshape(_B, _V)
    grid = (_B // _ROWS_PER_BLOCK,)
    p2d, act = pl.pallas_call(
        _sample_kernel,
        grid=grid,
        in_specs=[
            pl.BlockSpec((_ROWS_PER_BLOCK, _V), lambda i: (i, 0)),
            pl.BlockSpec((_ROWS_PER_BLOCK, _V), lambda i: (i, 0)),
        ],
        out_specs=[
            pl.BlockSpec((_ROWS_PER_BLOCK, _V), lambda i: (i, 0)),
            pl.BlockSpec((_ROWS_PER_BLOCK, 1), lambda i: (i, 0)),
        ],
        out_shape=[
            jax.ShapeDtypeStruct((_B, _V), jnp.float32),
            jax.ShapeDtypeStruct((_B, 1), jnp.int32),
        ],
        compiler_params=pltpu.CompilerParams(
            dimension_semantics=("parallel",),
        ),
    )(x2d, w)
    return p2d.reshape(_B, 1, _V), act


_W_DEV = None


def kernel(probs):
    global _W_DEV
    if _W_DEV is None:
        _W_DEV = jax.device_put(jnp.asarray(_W))
    return _run(probs, _W_DEV)


# speedup vs baseline: 4.1093x; 2.0766x over previous
import numpy as np

import jax
import jax.numpy as jnp
from jax.experimental import pallas as pl
from jax.experimental.pallas import tpu as pltpu

_B = 128
_V = 100000
_VB_SUM = 25000
_VB_MAIN = 10000


def _gumbel_weights_np(n):
    i = np.arange(n, dtype=np.uint32)
    k1 = np.uint32(0)
    k2 = np.uint32(42)
    k3 = k1 ^ k2 ^ np.uint32(0x1BD11BDA)
    ks = (k1, k2, k3)
    rot_a = (13, 15, 26, 6)
    rot_b = (17, 29, 16, 24)

    def rotl(x, d):
        return (x << np.uint32(d)) | (x >> np.uint32(32 - d))

    def four_rounds(x0, x1, rots):
        for r in rots:
            x0 = x0 + x1
            x1 = x0 ^ rotl(x1, r)
        return x0, x1

    with np.errstate(over="ignore"):
        x0 = np.zeros(n, np.uint32) + ks[0]
        x1 = i + ks[1]
        x0, x1 = four_rounds(x0, x1, rot_a)
        x0 = x0 + ks[1]
        x1 = x1 + ks[2] + np.uint32(1)
        x0, x1 = four_rounds(x0, x1, rot_b)
        x0 = x0 + ks[2]
        x1 = x1 + ks[0] + np.uint32(2)
        x0, x1 = four_rounds(x0, x1, rot_a)
        x0 = x0 + ks[0]
        x1 = x1 + ks[1] + np.uint32(3)
        x0, x1 = four_rounds(x0, x1, rot_b)
        x0 = x0 + ks[1]
        x1 = x1 + ks[2] + np.uint32(4)
        x0, x1 = four_rounds(x0, x1, rot_a)
        x0 = x0 + ks[2]
        x1 = x1 + ks[0] + np.uint32(5)
    bits = x0 ^ x1

    tiny = np.float32(np.finfo(np.float32).tiny)
    fb = (bits >> np.uint32(9)) | np.uint32(0x3F800000)
    f = fb.view(np.float32) - np.float32(1.0)
    u = np.maximum(tiny, f * (np.float32(1.0) - tiny) + tiny)
    w = -1.0 / np.log(u.astype(np.float64))
    return w.astype(np.float32)


_WT = np.ascontiguousarray(_gumbel_weights_np(_B * _V).reshape(_B, _V).T)


def _sum_kernel(x_ref, s_ref):
    step = pl.program_id(0)

    @pl.when(step == 0)
    def _init():
        s_ref[...] = jnp.zeros_like(s_ref)

    s_ref[...] += jnp.sum(x_ref[...], axis=0, keepdims=True)


def _main_kernel(s_ref, x_ref, w_ref, p_ref, act_ref, best_ref, bidx_ref):
    step = pl.program_id(0)
    nsteps = pl.num_programs(0)

    @pl.when(step == 0)
    def _init():
        best_ref[...] = jnp.full_like(best_ref, -jnp.inf)
        bidx_ref[...] = jnp.zeros_like(bidx_ref)

    x = x_ref[...]

    s = s_ref[...]
    s = jnp.where(s == 0.0, 1.0, s)
    p_ref[...] = x * (1.0 / s)

    t = x * w_ref[...]
    m = jnp.max(t, axis=0, keepdims=True)
    ri = jax.lax.broadcasted_iota(jnp.int32, t.shape, 0)
    cand = jnp.where(t == m, ri, jnp.int32(_V))
    idx = jnp.min(cand, axis=0, keepdims=True) + step * _VB_MAIN

    upd = m > best_ref[...]
    bidx_ref[...] = jnp.where(upd, idx, bidx_ref[...])
    best_ref[...] = jnp.maximum(best_ref[...], m)

    @pl.when(step == nsteps - 1)
    def _emit():
        act_ref[...] = bidx_ref[...]


@jax.jit
def _run(probs, wt):
    xt = probs.reshape(_B, _V).T

    sums = pl.pallas_call(
        _sum_kernel,
        grid=(_V // _VB_SUM,),
        in_specs=[pl.BlockSpec((_VB_SUM, _B), lambda i: (i, 0))],
        out_specs=pl.BlockSpec((1, _B), lambda i: (0, 0)),
        out_shape=jax.ShapeDtypeStruct((1, _B), jnp.float32),
    )(xt)

    pt, act = pl.pallas_call(
        _main_kernel,
        grid=(_V // _VB_MAIN,),
        in_specs=[
            pl.BlockSpec((1, _B), lambda i: (0, 0)),
            pl.BlockSpec((_VB_MAIN, _B), lambda i: (i, 0)),
            pl.BlockSpec((_VB_MAIN, _B), lambda i: (i, 0)),
        ],
        out_specs=[
            pl.BlockSpec((_VB_MAIN, _B), lambda i: (i, 0)),
            pl.BlockSpec((1, _B), lambda i: (0, 0)),
        ],
        out_shape=[
            jax.ShapeDtypeStruct((_V, _B), jnp.float32),
            jax.ShapeDtypeStruct((1, _B), jnp.int32),
        ],
        scratch_shapes=[
            pltpu.VMEM((1, _B), jnp.float32),
            pltpu.VMEM((1, _B), jnp.int32),
        ],
    )(sums, xt, wt)

    p = pt.T.reshape(_B, 1, _V)
    return p, act.reshape(_B, 1)


_WT_DEV = None


def kernel(probs):
    global _WT_DEV
    if _WT_DEV is None:
        _WT_DEV = jax.device_put(jnp.asarray(_WT))
    return _run(probs, _WT_DEV)
